# baseline (speedup 1.0000x reference)
import jax
import jax.numpy as jnp
import numpy as np
from jax import lax
from jax.experimental import pallas as pl
from jax.experimental.pallas import tpu as pltpu
from jax.experimental.pallas import tpu_sc as plsc

NODE_DIM = 128
N_GRAPHS = 512
N_NODES = 100000
TILE = 20000
LOG2 = float(np.log(2.0))
LOG2E = float(np.log2(np.e))

NW = 32
N_PAD = 100352
CHUNK = N_PAD // NW
REPS = CHUNK // 16


def _mlp_body(x_ref, w1_ref, b1_ref, w2_ref, s_ref):
    xb = x_ref[...]
    h = jnp.dot(xb, w1_ref[...], preferred_element_type=jnp.float32)
    h = h + b1_ref[...]
    e = jnp.exp2(jnp.minimum(h, -h) * LOG2E)
    sp = jnp.maximum(h, 0.0) + LOG2 * jnp.log2(1.0 + e)
    s = jnp.dot(sp, w2_ref[...], preferred_element_type=jnp.float32)
    s_ref[...] = s - LOG2 * jnp.sum(w2_ref[...])


def _seg_sum_sc(s_hbm, b_hbm, out_hbm, s_v, i_v, acc, red, tmp, shared):
    cid = lax.axis_index("c")
    sid = lax.axis_index("s")
    wid = sid * 2 + cid
    base = wid * CHUNK
    pltpu.sync_copy(s_hbm.at[pl.ds(base, CHUNK)], s_v)
    pltpu.sync_copy(b_hbm.at[pl.ds(base, CHUNK)], i_v)
    for j in range(N_GRAPHS // 16):
        acc[pl.ds(16 * j, 16)] = jnp.zeros((16,), jnp.float32)

    def body(j, carry):
        sv = s_v[pl.ds(j * 16, 16)]
        iv = i_v[pl.ds(j * 16, 16)]
        plsc.addupdate_scatter(acc, [iv], sv)
        return carry

    lax.fori_loop(0, REPS, body, 0)
    pltpu.sync_copy(acc, shared.at[sid])
    plsc.subcore_barrier()
    for half in range(2):
        col = sid * 32 + half * 16
        tot = jnp.zeros((16,), jnp.float32)
        for k in range(16):
            pltpu.sync_copy(shared.at[k, pl.ds(col, 16)], tmp)
            tot = tot + tmp[...]
        red[pl.ds(half * 16, 16)] = tot
    pltpu.sync_copy(red, out_hbm.at[cid, pl.ds(sid * 32, 32)])


def _tc_stage(x, W1, b1r, W2):
    nb = N_NODES // TILE
    return pl.pallas_call(
        _mlp_body,
        grid=(nb,),
        in_specs=[
            pl.BlockSpec((TILE, NODE_DIM), lambda i: (i, 0)),
            pl.BlockSpec((NODE_DIM, NODE_DIM), lambda i: (0, 0)),
            pl.BlockSpec((1, NODE_DIM), lambda i: (0, 0)),
            pl.BlockSpec((NODE_DIM, 1), lambda i: (0, 0)),
        ],
        out_specs=pl.BlockSpec((TILE, 1), lambda i: (i, 0)),
        out_shape=jax.ShapeDtypeStruct((N_NODES, 1), jnp.float32),
    )(x, W1, b1r, W2)


@jax.jit
def _run(x, b_pad, W1, b1r, W2):
    s = _tc_stage(x, W1, b1r, W2)
    s_pad = jnp.pad(s.reshape(-1), (0, N_PAD - N_NODES))

    mesh = plsc.VectorSubcoreMesh(core_axis_name="c", subcore_axis_name="s")
    kern = pl.kernel(
        _seg_sum_sc,
        mesh=mesh,
        out_type=jax.ShapeDtypeStruct((2, N_GRAPHS), jnp.float32),
        scratch_types=[
            pltpu.VMEM((CHUNK,), jnp.float32),
            pltpu.VMEM((CHUNK,), jnp.int32),
            pltpu.VMEM((N_GRAPHS,), jnp.float32),
            pltpu.VMEM((32,), jnp.float32),
            pltpu.VMEM((16,), jnp.float32),
            pltpu.VMEM_SHARED((16, N_GRAPHS), jnp.float32),
        ],
        compiler_params=pltpu.CompilerParams(needs_layout_passes=False),
    )
    parts = kern(s_pad, b_pad)
    return jnp.sum(parts, axis=0).reshape(N_GRAPHS, 1)


def kernel(x, batch, W1, b1, W2):
    b32 = batch.astype(jnp.int32)
    b_pad = jnp.pad(b32, (0, N_PAD - N_NODES))
    b1r = b1.reshape(1, NODE_DIM)
    return _run(x, b_pad, W1, b1r, W2)


# speedup vs baseline: 1.8859x; 1.8859x over previous
import jax
import jax.numpy as jnp
import numpy as np
from jax.experimental import pallas as pl

NODE_DIM = 128
N_GRAPHS = 512
N_NODES = 100000
TILE = 20000
NQ = 32
NR = 16
LOG2 = float(np.log(2.0))


def _fused_body(x_ref, seg_ref, w1_ref, b1_ref, w2_ref, out_ref):
    xb = x_ref[...]
    h = jnp.dot(xb, w1_ref[...], preferred_element_type=jnp.float32)
    h = h + b1_ref[...]
    h = jnp.maximum(h, 0.0) + jnp.log(1.0 + jnp.exp(jnp.minimum(h, -h))) - LOG2
    s = jnp.sum(h * w2_ref[...], axis=1, keepdims=True)

    seg_row = seg_ref[0]
    q_row = seg_row >> 4
    r_row = seg_row & 15
    r_col = r_row.reshape(TILE, 1)
    a = (jax.lax.broadcasted_iota(jnp.int32, (NQ, TILE), 0) == q_row)
    b = (jax.lax.broadcasted_iota(jnp.int32, (TILE, NR), 1) == r_col)
    bs = b.astype(jnp.float32) * s
    out_ref[0] = jnp.dot(a.astype(jnp.float32), bs,
                         preferred_element_type=jnp.float32)


@jax.jit
def _run(x, seg_r, W1, b1r, w2r):
    nb = N_NODES // TILE
    parts = pl.pallas_call(
        _fused_body,
        grid=(nb,),
        in_specs=[
            pl.BlockSpec((TILE, NODE_DIM), lambda i: (i, 0)),
            pl.BlockSpec((1, 1, TILE), lambda i: (i, 0, 0)),
            pl.BlockSpec((NODE_DIM, NODE_DIM), lambda i: (0, 0)),
            pl.BlockSpec((1, NODE_DIM), lambda i: (0, 0)),
            pl.BlockSpec((1, NODE_DIM), lambda i: (0, 0)),
        ],
        out_specs=pl.BlockSpec((1, NQ, NR), lambda i: (i, 0, 0)),
        out_shape=jax.ShapeDtypeStruct((nb, NQ, NR), jnp.float32),
    )(x, seg_r, W1, b1r, w2r)
    return jnp.sum(parts, axis=0).reshape(N_GRAPHS, 1)


def kernel(x, batch, W1, b1, W2):
    nb = N_NODES // TILE
    seg_r = batch.astype(jnp.int32).reshape(nb, 1, TILE)
    b1r = b1.reshape(1, NODE_DIM)
    w2r = W2.reshape(1, NODE_DIM)
    return _run(x, seg_r, W1, b1r, w2r)


# speedup vs baseline: 2.0895x; 1.1080x over previous
import jax
import jax.numpy as jnp
import numpy as np
from jax.experimental import pallas as pl

NODE_DIM = 128
N_GRAPHS = 512
N_NODES = 100000
TILE = 20000
NQ = 32
NR = 16
LOG2 = float(np.log(2.0))


def _fused_body(x_ref, seg_ref, w1_ref, b1_ref, w2_ref, out_ref):
    xb = x_ref[...]
    h = jnp.dot(xb, w1_ref[...], preferred_element_type=jnp.float32)
    h = h.astype(jnp.bfloat16) + b1_ref[...]
    h = jnp.maximum(h, 0.0) + jnp.log(1.0 + jnp.exp(jnp.minimum(h, -h))) - LOG2
    s = jnp.sum((h * w2_ref[...]).astype(jnp.float32), axis=1, keepdims=True)

    seg_row = seg_ref[0]
    q_row = seg_row >> 4
    r_row = seg_row & 15
    r_col = r_row.reshape(TILE, 1)
    a = (jax.lax.broadcasted_iota(jnp.int32, (NQ, TILE), 0) == q_row)
    b = (jax.lax.broadcasted_iota(jnp.int32, (TILE, NR), 1) == r_col)
    bs = jnp.where(b, s, 0.0)
    out_ref[0] = jnp.dot(a.astype(jnp.float32), bs,
                         preferred_element_type=jnp.float32)


@jax.jit
def _run(x, seg_r, W1, b1r, w2r):
    nb = N_NODES // TILE
    parts = pl.pallas_call(
        _fused_body,
        grid=(nb,),
        in_specs=[
            pl.BlockSpec((TILE, NODE_DIM), lambda i: (i, 0)),
            pl.BlockSpec((1, 1, TILE), lambda i: (i, 0, 0)),
            pl.BlockSpec((NODE_DIM, NODE_DIM), lambda i: (0, 0)),
            pl.BlockSpec((1, NODE_DIM), lambda i: (0, 0)),
            pl.BlockSpec((1, NODE_DIM), lambda i: (0, 0)),
        ],
        out_specs=pl.BlockSpec((1, NQ, NR), lambda i: (i, 0, 0)),
        out_shape=jax.ShapeDtypeStruct((nb, NQ, NR), jnp.float32),
    )(x, seg_r, W1, b1r, w2r)
    return jnp.sum(parts, axis=0).reshape(N_GRAPHS, 1)


def kernel(x, batch, W1, b1, W2):
    nb = N_NODES // TILE
    seg_r = batch.astype(jnp.int32).reshape(nb, 1, TILE)
    b1r = b1.reshape(1, NODE_DIM).astype(jnp.bfloat16)
    w2r = W2.reshape(1, NODE_DIM).astype(jnp.bfloat16)
    return _run(x, seg_r, W1, b1r, w2r)
